# Initial kernel scaffold; baseline (speedup 1.0000x reference)
import jax
import jax.numpy as jnp
from jax.experimental import pallas as pl


def kernel(x, batch, edge_index, edge_weight, W1, b1, W2, b2, Wp, bp, gamma, beta, alpha):
    raise NotImplementedError("write your pallas kernel here")


# speedup vs baseline: 5.1227x; 5.1227x over previous
import functools
import jax
import jax.numpy as jnp
from jax import lax
from jax.experimental import pallas as pl
from jax.experimental.pallas import tpu as pltpu
from jax.experimental.pallas import tpu_sc as plsc

NC = 2
NS = 16
NW = NC * NS
LN = 16
C_EDGE = 128
ROW_CHUNK = 128
GRAPHS = 128


def _ceil_to(v, m):
    return ((v + m - 1) // m) * m


def _zeros16():
    return jnp.zeros((LN,), jnp.float32)


def _qrsqrt16(v):
    i = lax.bitcast_convert_type(v, jnp.int32)
    i = jnp.int32(0x5F3759DF) - lax.shift_right_arithmetic(i, 1)
    y = lax.bitcast_convert_type(i, jnp.float32)
    for _ in range(3):
        y = y * (jnp.float32(1.5) - jnp.float32(0.5) * v * y * y)
    return y


def _make_coef_kernel(n_pad, e1p, e2p):
    mesh = plsc.VectorSubcoreMesh(core_axis_name="c", subcore_axis_name="s",
                                  num_cores=NC, num_subcores=NS)
    nz = n_pad // NS

    def body(src1, dst1, ew1, src2, dst2, ew2, coef1, coef2,
             deg1_s, deg2_s, sidx_v, didx_v, ew_v, cf_v, sv_v, dv_v, sl_v):
        s = lax.axis_index("s")
        c = lax.axis_index("c")
        w = c * NS + s

        def zb_body(i, _):
            sl_v[pl.ds(i * LN, LN)] = _zeros16()
            return 0
        lax.fori_loop(0, nz // LN, zb_body, 0)
        pltpu.sync_copy(sl_v, deg1_s.at[pl.ds(s * nz, nz)])
        pltpu.sync_copy(sl_v, deg2_s.at[pl.ds(s * nz, nz)])
        plsc.subcore_barrier()

        def deg_loop(deg_s, dst_hbm, ew_hbm, ept):
            def chunk(k, _):
                off = s * ept + k * C_EDGE
                pltpu.sync_copy(dst_hbm.at[pl.ds(off, C_EDGE)], didx_v)
                pltpu.sync_copy(ew_hbm.at[pl.ds(off, C_EDGE)], ew_v)
                pltpu.sync_copy(ew_v, deg_s.at[didx_v], add=True)
                return 0
            lax.fori_loop(0, ept // C_EDGE, chunk, 0)
        deg_loop(deg1_s, dst1, ew1, e1p // NS)
        deg_loop(deg2_s, dst2, ew2, e2p // NS)
        plsc.subcore_barrier()

        for deg_s in (deg1_s, deg2_s):
            pltpu.sync_copy(deg_s.at[pl.ds(s * nz, nz)], sl_v)

            def rs_body(i, _):
                sl = pl.ds(i * LN, LN)
                v = jnp.maximum(sl_v[sl], jnp.float32(1e-6))
                sl_v[sl] = _qrsqrt16(v)
                return 0
            lax.fori_loop(0, nz // LN, rs_body, 0)
            pltpu.sync_copy(sl_v, deg_s.at[pl.ds(s * nz, nz)])
        plsc.subcore_barrier()

        def coef_loop(src_hbm, dst_hbm, ew_hbm, coef_hbm, dinv_s, ept):
            def chunk(k, _):
                off = w * ept + k * C_EDGE
                pltpu.sync_copy(src_hbm.at[pl.ds(off, C_EDGE)], sidx_v)
                pltpu.sync_copy(dst_hbm.at[pl.ds(off, C_EDGE)], didx_v)
                pltpu.sync_copy(ew_hbm.at[pl.ds(off, C_EDGE)], ew_v)
                pltpu.sync_copy(dinv_s.at[sidx_v], sv_v)
                pltpu.sync_copy(dinv_s.at[didx_v], dv_v)
                for g in range(C_EDGE // LN):
                    sl = pl.ds(g * LN, LN)
                    cf_v[sl] = ew_v[sl] * sv_v[sl] * dv_v[sl]
                pltpu.sync_copy(cf_v, coef_hbm.at[pl.ds(off, C_EDGE)])
                return 0
            lax.fori_loop(0, ept // C_EDGE, chunk, 0)
        coef_loop(src1, dst1, ew1, coef1, deg1_s, e1p // NW)
        coef_loop(src2, dst2, ew2, coef2, deg2_s, e2p // NW)

    return pl.kernel(
        body,
        out_type=[jax.ShapeDtypeStruct((e1p,), jnp.float32),
                  jax.ShapeDtypeStruct((e2p,), jnp.float32)],
        mesh=mesh,
        scratch_types=[
            pltpu.VMEM_SHARED((n_pad,), jnp.float32),
            pltpu.VMEM_SHARED((n_pad,), jnp.float32),
            pltpu.VMEM((C_EDGE,), jnp.int32),
            pltpu.VMEM((C_EDGE,), jnp.int32),
            pltpu.VMEM((C_EDGE,), jnp.float32),
            pltpu.VMEM((C_EDGE,), jnp.float32),
            pltpu.VMEM((C_EDGE,), jnp.float32),
            pltpu.VMEM((C_EDGE,), jnp.float32),
            pltpu.VMEM((nz,), jnp.float32),
        ],
    )


def _make_conv_kernel(n_pad, h, e1p, e2p, pool):
    mesh = plsc.VectorSubcoreMesh(core_axis_name="c", subcore_axis_name="s",
                                  num_cores=NC, num_subcores=NS)
    rpt = n_pad // NS
    g_per_tile = GRAPHS // NS

    def body(*refs):
        if pool:
            (t1, t2, src1, dst1, coef1, src2, dst2, coef2, batch_hbm,
             o1, o2, acc_s, g_s, rows_v, sidx_v, didx_v, cf_v, zrow_v,
             sem) = refs
        else:
            (t1, t2, src1, dst1, coef1, src2, dst2, coef2,
             o1, o2, acc_s, rows_v, sidx_v, didx_v, cf_v, zrow_v,
             sem) = refs
        s = lax.axis_index("s")
        c = lax.axis_index("c")
        w = c * NS + s

        def zr_body(i, _):
            for f in range(h // LN):
                zrow_v[i, pl.ds(f * LN, LN)] = _zeros16()
            return 0
        lax.fori_loop(0, ROW_CHUNK, zr_body, 0)

        def stage(table, src, dst, coef, out, ept):
            def zacc(i, _):
                pltpu.sync_copy(
                    zrow_v, acc_s.at[pl.ds(s * rpt + i * ROW_CHUNK, ROW_CHUNK)])
                return 0
            lax.fori_loop(0, rpt // ROW_CHUNK, zacc, 0)
            plsc.subcore_barrier()

            def chunk(k, _):
                off = w * ept + k * C_EDGE
                pltpu.sync_copy(src.at[pl.ds(off, C_EDGE)], sidx_v)
                cp = pltpu.async_copy(table.at[sidx_v], rows_v, sem)
                pltpu.sync_copy(dst.at[pl.ds(off, C_EDGE)], didx_v)
                pltpu.sync_copy(coef.at[pl.ds(off, C_EDGE)], cf_v)
                cp.wait()

                def scale(g, _):
                    cfv = cf_v[pl.ds(g * LN, LN)]
                    for j in range(LN):
                        cfb = lax.gather(
                            cfv, jnp.full((LN, 1), j, jnp.int32),
                            lax.GatherDimensionNumbers(
                                offset_dims=(), collapsed_slice_dims=(0,),
                                start_index_map=(0,)),
                            slice_sizes=(1,),
                            mode=lax.GatherScatterMode.PROMISE_IN_BOUNDS)
                        e = g * LN + j
                        for f in range(h // LN):
                            sl = pl.ds(f * LN, LN)
                            rows_v[e, sl] = rows_v[e, sl] * cfb
                    return 0
                lax.fori_loop(0, C_EDGE // LN, scale, 0)
                pltpu.sync_copy(rows_v, acc_s.at[didx_v], add=True)
                return 0
            lax.fori_loop(0, ept // C_EDGE, chunk, 0)
            plsc.subcore_barrier()

            if pool:
                pltpu.sync_copy(zrow_v.at[pl.ds(0, g_per_tile)],
                                g_s.at[pl.ds(s * g_per_tile, g_per_tile)])
                plsc.subcore_barrier()

                def poolc(i, _):
                    r0 = s * rpt + i * ROW_CHUNK
                    pltpu.sync_copy(acc_s.at[pl.ds(r0, ROW_CHUNK)], rows_v)
                    pltpu.sync_copy(batch_hbm.at[pl.ds(r0, ROW_CHUNK)], didx_v)
                    pltpu.sync_copy(rows_v, g_s.at[didx_v], add=True)
                    return 0
                lax.fori_loop(0, rpt // ROW_CHUNK, poolc, 0)
                plsc.subcore_barrier()
                pltpu.sync_copy(
                    g_s.at[pl.ds(s * g_per_tile, g_per_tile)],
                    out.at[pl.ds(c * GRAPHS + s * g_per_tile, g_per_tile)])
                plsc.subcore_barrier()
            else:
                def wb(i, _):
                    r0 = s * rpt + i * ROW_CHUNK
                    pltpu.sync_copy(acc_s.at[pl.ds(r0, ROW_CHUNK)],
                                    out.at[pl.ds(c * n_pad + r0, ROW_CHUNK)])
                    return 0
                lax.fori_loop(0, rpt // ROW_CHUNK, wb, 0)

        stage(t1, src1, dst1, coef1, o1, e1p // NW)
        stage(t2, src2, dst2, coef2, o2, e2p // NW)

    out_rows = NC * (GRAPHS if pool else n_pad)
    scratch = [
        pltpu.VMEM_SHARED((n_pad, h), jnp.float32),
    ]
    if pool:
        scratch.append(pltpu.VMEM_SHARED((GRAPHS, h), jnp.float32))
    scratch += [
        pltpu.VMEM((C_EDGE, h), jnp.float32),
        pltpu.VMEM((C_EDGE,), jnp.int32),
        pltpu.VMEM((C_EDGE,), jnp.int32),
        pltpu.VMEM((C_EDGE,), jnp.float32),
        pltpu.VMEM((ROW_CHUNK, h), jnp.float32),
        pltpu.SemaphoreType.DMA,
    ]
    return pl.kernel(
        body,
        out_type=[jax.ShapeDtypeStruct((out_rows, h), jnp.float32),
                  jax.ShapeDtypeStruct((out_rows, h), jnp.float32)],
        mesh=mesh,
        scratch_types=scratch,
    )


def _mm_dual(x_p, w1, fm, b1, bn):
    n_pad, d = x_p.shape
    h = w1.shape[1]

    def body(x_ref, w_ref, fm_ref, b_ref, o1_ref, o2_ref):
        xb = x_ref[...]
        wv = w_ref[...]
        bv = b_ref[...]
        o1_ref[...] = jnp.dot(xb, wv, preferred_element_type=jnp.float32) + bv
        o2_ref[...] = jnp.dot(xb * fm_ref[...], wv,
                              preferred_element_type=jnp.float32) + bv

    grid = (n_pad // bn,)
    return pl.pallas_call(
        body,
        grid=grid,
        in_specs=[
            pl.BlockSpec((bn, d), lambda i: (i, 0)),
            pl.BlockSpec((d, h), lambda i: (0, 0)),
            pl.BlockSpec((1, d), lambda i: (0, 0)),
            pl.BlockSpec((1, h), lambda i: (0, 0)),
        ],
        out_specs=[pl.BlockSpec((bn, h), lambda i: (i, 0)),
                   pl.BlockSpec((bn, h), lambda i: (i, 0))],
        out_shape=[jax.ShapeDtypeStruct((n_pad, h), jnp.float32),
                   jax.ShapeDtypeStruct((n_pad, h), jnp.float32)],
    )(x_p, w1, fm, b1)


def _relu_mm(ap, w2, b2, bn):
    two_n, h = ap.shape
    n_pad = two_n // 2
    nb = n_pad // bn

    def body(lo_ref, hi_ref, w_ref, b_ref, o_ref):
        hv = jnp.maximum(lo_ref[...] + hi_ref[...], 0.0)
        o_ref[...] = jnp.dot(hv, w_ref[...],
                             preferred_element_type=jnp.float32) + b_ref[...]

    return pl.pallas_call(
        body,
        grid=(nb,),
        in_specs=[
            pl.BlockSpec((bn, h), lambda i: (i, 0)),
            pl.BlockSpec((bn, h), lambda i, _nb=nb: (i + _nb, 0)),
            pl.BlockSpec((h, h), lambda i: (0, 0)),
            pl.BlockSpec((1, h), lambda i: (0, 0)),
        ],
        out_specs=pl.BlockSpec((bn, h), lambda i: (i, 0)),
        out_shape=jax.ShapeDtypeStruct((n_pad, h), jnp.float32),
    )(ap, ap, w2, b2)


def _predictor(g1p, g2p, wp, bp, gm, bt, al):
    h = wp.shape[0]

    def body(g1_ref, g2_ref, w_ref, b_ref, gm_ref, bt_ref, al_ref,
             o1_ref, o2_ref, p1_ref, p2_ref):
        wv = w_ref[...]
        bv = b_ref[...]
        gmv = gm_ref[...]
        btv = bt_ref[...]
        alv = al_ref[0, 0]

        def pred(g):
            y = jnp.dot(g, wv, preferred_element_type=jnp.float32) + bv
            mu = jnp.mean(y, axis=0, keepdims=True)
            dlt = y - mu
            var = jnp.mean(dlt * dlt, axis=0, keepdims=True)
            yb = gmv * (dlt * lax.rsqrt(var + 1e-5)) + btv
            return jnp.where(yb > 0, yb, alv * yb)

        a1 = g1_ref[...]
        a2 = g2_ref[...]
        g1 = a1[:GRAPHS] + a1[GRAPHS:]
        g2 = a2[:GRAPHS] + a2[GRAPHS:]
        o1_ref[...] = g1
        o2_ref[...] = g2
        p1_ref[...] = pred(g1)
        p2_ref[...] = pred(g2)

    out = jax.ShapeDtypeStruct((GRAPHS, h), jnp.float32)
    return pl.pallas_call(
        body,
        out_shape=[out, out, out, out],
    )(g1p, g2p, wp, bp, gm, bt, al)


def kernel(x, batch, edge_index, edge_weight, W1, b1, W2, b2, Wp, bp,
           gamma, beta, alpha):
    n, d = x.shape
    h = W1.shape[1]
    e = edge_weight.shape[0]

    src1, dst1 = edge_index[0], edge_index[1]
    src2, dst2, ew2 = src1[::2], dst1[::2], edge_weight[::2]
    e2 = ew2.shape[0]

    align = NW * C_EDGE
    e1p = _ceil_to(e, align)
    e2p = _ceil_to(e2, align)
    n_pad = _ceil_to(n, NS * ROW_CHUNK)

    pad1 = e1p - e
    pad2 = e2p - e2
    src1 = jnp.pad(src1, (0, pad1))
    dst1 = jnp.pad(dst1, (0, pad1))
    ew1 = jnp.pad(edge_weight, (0, pad1))
    src2 = jnp.pad(src2, (0, pad2))
    dst2 = jnp.pad(dst2, (0, pad2))
    ew2 = jnp.pad(ew2, (0, pad2))

    x_p = jnp.pad(x, ((0, n_pad - n), (0, 0)))
    batch_p = jnp.pad(batch, (0, n_pad - n))

    fmask = jax.random.bernoulli(jax.random.key(7), 0.8, (1, d)).astype(
        jnp.float32)
    b1r = b1.reshape(1, h)
    b2r = b2.reshape(1, h)
    bpr = bp.reshape(1, h)
    gmr = gamma.reshape(1, h)
    btr = beta.reshape(1, h)
    alr = alpha.reshape(1, 1)

    bn = 1024

    xw1, xw2 = _mm_dual(x_p, W1, fmask, b1r, bn)
    coef1, coef2 = _make_coef_kernel(n_pad, e1p, e2p)(
        src1, dst1, ew1, src2, dst2, ew2)
    a1p, a2p = _make_conv_kernel(n_pad, h, e1p, e2p, pool=False)(
        xw1, xw2, src1, dst1, coef1, src2, dst2, coef2)
    hw1 = _relu_mm(a1p, W2, b2r, bn)
    hw2 = _relu_mm(a2p, W2, b2r, bn)
    g1p, g2p = _make_conv_kernel(n_pad, h, e1p, e2p, pool=True)(
        hw1, hw2, src1, dst1, coef1, src2, dst2, coef2, batch_p)
    g1, g2, p1, p2 = _predictor(g1p, g2p, Wp, bpr, gmr, btr, alr)

    return (g1, g2, p1, p2, g1, g2)


# speedup vs baseline: 6.3200x; 1.2337x over previous
import functools
import jax
import jax.numpy as jnp
from jax import lax
from jax.experimental import pallas as pl
from jax.experimental.pallas import tpu as pltpu
from jax.experimental.pallas import tpu_sc as plsc

NC = 2
NS = 16
NW = NC * NS
LN = 16
C_EDGE = 128
ROW_CHUNK = 128
ZROWS = 64
GRAPHS = 128


def _ceil_to(v, m):
    return ((v + m - 1) // m) * m


def _zeros16():
    return jnp.zeros((LN,), jnp.float32)


def _qrsqrt16(v):
    i = lax.bitcast_convert_type(v, jnp.int32)
    i = jnp.int32(0x5F3759DF) - lax.shift_right_arithmetic(i, 1)
    y = lax.bitcast_convert_type(i, jnp.float32)
    for _ in range(3):
        y = y * (jnp.float32(1.5) - jnp.float32(0.5) * v * y * y)
    return y


def _make_coef_kernel(n_pad, e1p, e2p):
    mesh = plsc.VectorSubcoreMesh(core_axis_name="c", subcore_axis_name="s",
                                  num_cores=NC, num_subcores=NS)
    nz = n_pad // NS

    def body(meta1, meta2, coef1, coef2, deg1_s, deg2_s,
             m0, m1, ew0, ew1b, sv0, sv1, dv0, dv1, cf0, cf1, sl_v,
             msem0, msem1, gsem0, gsem1):
        s = lax.axis_index("s")
        c = lax.axis_index("c")
        w = c * NS + s
        mv = (m0, m1)
        ewv = (ew0, ew1b)
        svv = (sv0, sv1)
        dvv = (dv0, dv1)
        cfv = (cf0, cf1)
        msem = (msem0, msem1)
        gsem = (gsem0, gsem1)

        def zb_body(i, _):
            sl_v[pl.ds(i * LN, LN)] = _zeros16()
            return 0
        lax.fori_loop(0, nz // LN, zb_body, 0)
        pltpu.sync_copy(sl_v, deg1_s.at[pl.ds(s * nz, nz)])
        pltpu.sync_copy(sl_v, deg2_s.at[pl.ds(s * nz, nz)])
        plsc.subcore_barrier()

        def deg_pass(meta, deg_s, nch):
            base = s * nch

            def repack_and_scatter(b):
                for q in range(C_EDGE // LN):
                    sl = pl.ds(q * LN, LN)
                    ewv[b][sl] = lax.bitcast_convert_type(
                        mv[b][2, sl], jnp.float32)
                pltpu.sync_copy(ewv[b], deg_s.at[mv[b].at[1]], add=True)

            for b in range(2):
                pltpu.async_copy(meta.at[base + b], mv[b], msem[b])

            def grp(g, _):
                for b in range(2):
                    k = 2 * g + b
                    pltpu.make_async_copy(meta.at[base], mv[b],
                                          msem[b]).wait()
                    repack_and_scatter(b)
                    pltpu.async_copy(meta.at[base + k + 2], mv[b], msem[b])
                return 0
            lax.fori_loop(0, nch // 2 - 1, grp, 0)
            for b in range(2):
                pltpu.make_async_copy(meta.at[base], mv[b], msem[b]).wait()
                repack_and_scatter(b)
        deg_pass(meta1, deg1_s, e1p // C_EDGE // NS)
        deg_pass(meta2, deg2_s, e2p // C_EDGE // NS)
        plsc.subcore_barrier()

        for deg_s in (deg1_s, deg2_s):
            pltpu.sync_copy(deg_s.at[pl.ds(s * nz, nz)], sl_v)

            def rs_body(i, _):
                sl = pl.ds(i * LN, LN)
                v = jnp.maximum(sl_v[sl], jnp.float32(1e-6))
                sl_v[sl] = _qrsqrt16(v)
                return 0
            lax.fori_loop(0, nz // LN, rs_body, 0)
            pltpu.sync_copy(sl_v, deg_s.at[pl.ds(s * nz, nz)])
        plsc.subcore_barrier()

        def coef_pass(meta, coef_hbm, dinv_s, nch):
            base = w * nch

            def issue_gathers(b):
                pltpu.async_copy(dinv_s.at[mv[b].at[0]], svv[b], gsem[b])
                pltpu.async_copy(dinv_s.at[mv[b].at[1]], dvv[b], gsem[b])

            def wait_meta(b):
                pltpu.make_async_copy(meta.at[base], mv[b], msem[b]).wait()

            def wait_gathers(b):
                pltpu.make_async_copy(dinv_s.at[mv[b].at[0]], svv[b],
                                      gsem[b]).wait()
                pltpu.make_async_copy(dinv_s.at[mv[b].at[1]], dvv[b],
                                      gsem[b]).wait()

            def compute_store(b, k):
                for q in range(C_EDGE // LN):
                    sl = pl.ds(q * LN, LN)
                    ew = lax.bitcast_convert_type(mv[b][2, sl], jnp.float32)
                    cfv[b][sl] = ew * svv[b][sl] * dvv[b][sl]
                pltpu.sync_copy(cfv[b],
                                coef_hbm.at[pl.ds((base + k) * C_EDGE,
                                                  C_EDGE)])

            pltpu.async_copy(meta.at[base], mv[0], msem[0])
            pltpu.async_copy(meta.at[base + 1], mv[1], msem[1])
            wait_meta(0)
            issue_gathers(0)

            def grp(g, _):
                for b in range(2):
                    k = 2 * g + b
                    wait_gathers(b)
                    wait_meta(1 - b)
                    issue_gathers(1 - b)
                    compute_store(b, k)
                    pltpu.async_copy(meta.at[base + k + 2], mv[b], msem[b])
                return 0
            lax.fori_loop(0, nch // 2 - 1, grp, 0)
            wait_gathers(0)
            wait_meta(1)
            issue_gathers(1)
            compute_store(0, nch - 2)
            wait_gathers(1)
            compute_store(1, nch - 1)
        coef_pass(meta1, coef1, deg1_s, e1p // C_EDGE // NW)
        coef_pass(meta2, coef2, deg2_s, e2p // C_EDGE // NW)

    return pl.kernel(
        body,
        out_type=[jax.ShapeDtypeStruct((e1p,), jnp.float32),
                  jax.ShapeDtypeStruct((e2p,), jnp.float32)],
        mesh=mesh,
        scratch_types=[
            pltpu.VMEM_SHARED((n_pad,), jnp.float32),
            pltpu.VMEM_SHARED((n_pad,), jnp.float32),
            pltpu.VMEM((3, C_EDGE), jnp.int32),
            pltpu.VMEM((3, C_EDGE), jnp.int32),
            pltpu.VMEM((C_EDGE,), jnp.float32),
            pltpu.VMEM((C_EDGE,), jnp.float32),
            pltpu.VMEM((C_EDGE,), jnp.float32),
            pltpu.VMEM((C_EDGE,), jnp.float32),
            pltpu.VMEM((C_EDGE,), jnp.float32),
            pltpu.VMEM((C_EDGE,), jnp.float32),
            pltpu.VMEM((C_EDGE,), jnp.float32),
            pltpu.VMEM((C_EDGE,), jnp.float32),
            pltpu.VMEM((nz,), jnp.float32),
            pltpu.SemaphoreType.DMA,
            pltpu.SemaphoreType.DMA,
            pltpu.SemaphoreType.DMA,
            pltpu.SemaphoreType.DMA,
        ],
    )


def _make_conv_kernel(n_pad, h, e1p, e2p, pool):
    mesh = plsc.VectorSubcoreMesh(core_axis_name="c", subcore_axis_name="s",
                                  num_cores=NC, num_subcores=NS)
    rpt = n_pad // NS
    g_per_tile = GRAPHS // NS

    def body(*refs):
        if pool:
            (t1, t2, meta1, coef1, meta2, coef2, batch_hbm,
             o1, o2, acc_s, g_s, rows0, rows1, m0, m1, cf0, cf1, zrow_v,
             gsem0, gsem1, csem0, csem1) = refs
        else:
            (t1, t2, meta1, coef1, meta2, coef2,
             o1, o2, acc_s, rows0, rows1, m0, m1, cf0, cf1, zrow_v,
             gsem0, gsem1, csem0, csem1) = refs
        s = lax.axis_index("s")
        c = lax.axis_index("c")
        w = c * NS + s
        rows = (rows0, rows1)
        mv = (m0, m1)
        cfv = (cf0, cf1)
        gsem = (gsem0, gsem1)
        csem = (csem0, csem1)

        def zr_body(i, _):
            for f in range(h // LN):
                zrow_v[i, pl.ds(f * LN, LN)] = _zeros16()
            return 0
        lax.fori_loop(0, ZROWS, zr_body, 0)

        def stage(table, meta, coef, out, nch):
            base = w * nch

            def zacc(i, _):
                pltpu.sync_copy(
                    zrow_v, acc_s.at[pl.ds(s * rpt + i * ZROWS, ZROWS)])
                return 0
            lax.fori_loop(0, rpt // ZROWS, zacc, 0)
            plsc.subcore_barrier()

            def prefetch(b, k):
                pltpu.sync_copy(meta.at[base + k], mv[b])
                pltpu.async_copy(coef.at[pl.ds((base + k) * C_EDGE, C_EDGE)],
                                 cfv[b], csem[b])
                pltpu.async_copy(table.at[mv[b].at[0]], rows[b], gsem[b])

            def process(b):
                pltpu.make_async_copy(table.at[mv[b].at[0]], rows[b],
                                      gsem[b]).wait()
                pltpu.make_async_copy(coef.at[pl.ds(0, C_EDGE)], cfv[b],
                                      csem[b]).wait()

                def scale(g, _):
                    cfg = cfv[b][pl.ds(g * LN, LN)]
                    for j in range(LN):
                        cfb = lax.gather(
                            cfg, jnp.full((LN, 1), j, jnp.int32),
                            lax.GatherDimensionNumbers(
                                offset_dims=(), collapsed_slice_dims=(0,),
                                start_index_map=(0,)),
                            slice_sizes=(1,),
                            mode=lax.GatherScatterMode.PROMISE_IN_BOUNDS)
                        e = g * LN + j
                        for f in range(h // LN):
                            sl = pl.ds(f * LN, LN)
                            rows[b][e, sl] = rows[b][e, sl] * cfb
                    return 0
                lax.fori_loop(0, C_EDGE // LN, scale, 0)
                pltpu.sync_copy(rows[b], acc_s.at[mv[b].at[1]], add=True)

            for b in range(2):
                prefetch(b, b)

            def grp(g, _):
                for b in range(2):
                    k = 2 * g + b
                    process(b)
                    prefetch(b, k + 2)
                return 0
            lax.fori_loop(0, nch // 2 - 1, grp, 0)
            process(0)
            process(1)
            plsc.subcore_barrier()

            if pool:
                pltpu.sync_copy(zrow_v.at[pl.ds(0, g_per_tile)],
                                g_s.at[pl.ds(s * g_per_tile, g_per_tile)])
                plsc.subcore_barrier()

                def poolc(i, _):
                    r0 = s * rpt + i * ROW_CHUNK
                    pltpu.sync_copy(acc_s.at[pl.ds(r0, ROW_CHUNK)], rows0)
                    pltpu.sync_copy(batch_hbm.at[pl.ds(r0, ROW_CHUNK)],
                                    m0.at[1])
                    pltpu.sync_copy(rows0, g_s.at[m0.at[1]], add=True)
                    return 0
                lax.fori_loop(0, rpt // ROW_CHUNK, poolc, 0)
                plsc.subcore_barrier()
                pltpu.sync_copy(
                    g_s.at[pl.ds(s * g_per_tile, g_per_tile)],
                    out.at[pl.ds(c * GRAPHS + s * g_per_tile, g_per_tile)])
                plsc.subcore_barrier()
            else:
                def wb(i, _):
                    r0 = s * rpt + i * ROW_CHUNK
                    pltpu.sync_copy(acc_s.at[pl.ds(r0, ROW_CHUNK)],
                                    out.at[pl.ds(c * n_pad + r0, ROW_CHUNK)])
                    return 0
                lax.fori_loop(0, rpt // ROW_CHUNK, wb, 0)

        stage(t1, meta1, coef1, o1, e1p // C_EDGE // NW)
        stage(t2, meta2, coef2, o2, e2p // C_EDGE // NW)

    out_rows = NC * (GRAPHS if pool else n_pad)
    scratch = [
        pltpu.VMEM_SHARED((n_pad, h), jnp.float32),
    ]
    if pool:
        scratch.append(pltpu.VMEM_SHARED((GRAPHS, h), jnp.float32))
    scratch += [
        pltpu.VMEM((C_EDGE, h), jnp.float32),
        pltpu.VMEM((C_EDGE, h), jnp.float32),
        pltpu.VMEM((3, C_EDGE), jnp.int32),
        pltpu.VMEM((3, C_EDGE), jnp.int32),
        pltpu.VMEM((C_EDGE,), jnp.float32),
        pltpu.VMEM((C_EDGE,), jnp.float32),
        pltpu.VMEM((ZROWS, h), jnp.float32),
        pltpu.SemaphoreType.DMA,
        pltpu.SemaphoreType.DMA,
        pltpu.SemaphoreType.DMA,
        pltpu.SemaphoreType.DMA,
    ]
    return pl.kernel(
        body,
        out_type=[jax.ShapeDtypeStruct((out_rows, h), jnp.float32),
                  jax.ShapeDtypeStruct((out_rows, h), jnp.float32)],
        mesh=mesh,
        scratch_types=scratch,
    )


def _mm_dual(x_p, w1, fm, b1, bn):
    n_pad, d = x_p.shape
    h = w1.shape[1]

    def body(x_ref, w_ref, fm_ref, b_ref, o1_ref, o2_ref):
        xb = x_ref[...]
        wv = w_ref[...]
        bv = b_ref[...]
        o1_ref[...] = jnp.dot(xb, wv, preferred_element_type=jnp.float32) + bv
        o2_ref[...] = jnp.dot(xb * fm_ref[...], wv,
                              preferred_element_type=jnp.float32) + bv

    grid = (n_pad // bn,)
    return pl.pallas_call(
        body,
        grid=grid,
        in_specs=[
            pl.BlockSpec((bn, d), lambda i: (i, 0)),
            pl.BlockSpec((d, h), lambda i: (0, 0)),
            pl.BlockSpec((1, d), lambda i: (0, 0)),
            pl.BlockSpec((1, h), lambda i: (0, 0)),
        ],
        out_specs=[pl.BlockSpec((bn, h), lambda i: (i, 0)),
                   pl.BlockSpec((bn, h), lambda i: (i, 0))],
        out_shape=[jax.ShapeDtypeStruct((n_pad, h), jnp.float32),
                   jax.ShapeDtypeStruct((n_pad, h), jnp.float32)],
    )(x_p, w1, fm, b1)


def _relu_mm(ap, w2, b2, bn):
    two_n, h = ap.shape
    n_pad = two_n // 2
    nb = n_pad // bn

    def body(lo_ref, hi_ref, w_ref, b_ref, o_ref):
        hv = jnp.maximum(lo_ref[...] + hi_ref[...], 0.0)
        o_ref[...] = jnp.dot(hv, w_ref[...],
                             preferred_element_type=jnp.float32) + b_ref[...]

    return pl.pallas_call(
        body,
        grid=(nb,),
        in_specs=[
            pl.BlockSpec((bn, h), lambda i: (i, 0)),
            pl.BlockSpec((bn, h), lambda i, _nb=nb: (i + _nb, 0)),
            pl.BlockSpec((h, h), lambda i: (0, 0)),
            pl.BlockSpec((1, h), lambda i: (0, 0)),
        ],
        out_specs=pl.BlockSpec((bn, h), lambda i: (i, 0)),
        out_shape=jax.ShapeDtypeStruct((n_pad, h), jnp.float32),
    )(ap, ap, w2, b2)


def _predictor(g1p, g2p, wp, bp, gm, bt, al):
    h = wp.shape[0]

    def body(g1_ref, g2_ref, w_ref, b_ref, gm_ref, bt_ref, al_ref,
             o1_ref, o2_ref, p1_ref, p2_ref):
        wv = w_ref[...]
        bv = b_ref[...]
        gmv = gm_ref[...]
        btv = bt_ref[...]
        alv = al_ref[0, 0]

        def pred(g):
            y = jnp.dot(g, wv, preferred_element_type=jnp.float32) + bv
            mu = jnp.mean(y, axis=0, keepdims=True)
            dlt = y - mu
            var = jnp.mean(dlt * dlt, axis=0, keepdims=True)
            yb = gmv * (dlt * lax.rsqrt(var + 1e-5)) + btv
            return jnp.where(yb > 0, yb, alv * yb)

        a1 = g1_ref[...]
        a2 = g2_ref[...]
        g1 = a1[:GRAPHS] + a1[GRAPHS:]
        g2 = a2[:GRAPHS] + a2[GRAPHS:]
        o1_ref[...] = g1
        o2_ref[...] = g2
        p1_ref[...] = pred(g1)
        p2_ref[...] = pred(g2)

    out = jax.ShapeDtypeStruct((GRAPHS, h), jnp.float32)
    return pl.pallas_call(
        body,
        out_shape=[out, out, out, out],
    )(g1p, g2p, wp, bp, gm, bt, al)


def kernel(x, batch, edge_index, edge_weight, W1, b1, W2, b2, Wp, bp,
           gamma, beta, alpha):
    n, d = x.shape
    h = W1.shape[1]
    e = edge_weight.shape[0]

    src1, dst1 = edge_index[0], edge_index[1]
    src2, dst2, ew2 = src1[::2], dst1[::2], edge_weight[::2]
    e2 = ew2.shape[0]

    align = NW * C_EDGE * 2
    e1p = _ceil_to(e, align)
    e2p = _ceil_to(e2, align)
    n_pad = _ceil_to(n, NS * ROW_CHUNK)

    def pack_meta(src, dst, ew, ep):
        pad = ep - ew.shape[0]
        rows = jnp.stack([jnp.pad(src, (0, pad)),
                          jnp.pad(dst, (0, pad)),
                          lax.bitcast_convert_type(jnp.pad(ew, (0, pad)),
                                                   jnp.int32)], 0)
        return rows.reshape(3, ep // C_EDGE, C_EDGE).transpose(1, 0, 2)

    meta1 = pack_meta(src1, dst1, edge_weight, e1p)
    meta2 = pack_meta(src2, dst2, ew2, e2p)

    x_p = jnp.pad(x, ((0, n_pad - n), (0, 0)))
    batch_p = jnp.pad(batch, (0, n_pad - n))

    fmask = jax.random.bernoulli(jax.random.key(7), 0.8, (1, d)).astype(
        jnp.float32)
    b1r = b1.reshape(1, h)
    b2r = b2.reshape(1, h)
    bpr = bp.reshape(1, h)
    gmr = gamma.reshape(1, h)
    btr = beta.reshape(1, h)
    alr = alpha.reshape(1, 1)

    bn = 1024

    xw1, xw2 = _mm_dual(x_p, W1, fmask, b1r, bn)
    coef1, coef2 = _make_coef_kernel(n_pad, e1p, e2p)(meta1, meta2)
    a1p, a2p = _make_conv_kernel(n_pad, h, e1p, e2p, pool=False)(
        xw1, xw2, meta1, coef1, meta2, coef2)
    hw1 = _relu_mm(a1p, W2, b2r, bn)
    hw2 = _relu_mm(a2p, W2, b2r, bn)
    g1p, g2p = _make_conv_kernel(n_pad, h, e1p, e2p, pool=True)(
        hw1, hw2, meta1, coef1, meta2, coef2, batch_p)
    g1, g2, p1, p2 = _predictor(g1p, g2p, Wp, bpr, gmr, btr, alr)

    return (g1, g2, p1, p2, g1, g2)
